# Initial kernel scaffold; baseline (speedup 1.0000x reference)
import jax
import jax.numpy as jnp
from jax.experimental import pallas as pl


def kernel(xyz, curve1, curve2, curve3, W1, b1, W2, b2, W3, b3, Wfc, bfc):
    raise NotImplementedError("write your pallas kernel here")


# speedup vs baseline: 8.9048x; 8.9048x over previous
import functools

import jax
import jax.numpy as jnp
from jax import lax
from jax.experimental import pallas as pl
from jax.experimental.pallas import tpu as pltpu
from jax.experimental.pallas import tpu_sc as plsc

_B, _N = 32, 1024
_R = _B * _N
_NW = 32
_BN = 8

_SC_PARAMS = pltpu.CompilerParams(use_tc_tiling_on_sc=False)
_MESH = dict(core_axis_name="c", subcore_axis_name="s")


def _sc_stage1(xyzrows, gall, g1c, arc):
    mesh = plsc.VectorSubcoreMesh(**_MESH)

    @functools.partial(
        pl.kernel, mesh=mesh,
        out_type=(jax.ShapeDtypeStruct((3 * _R, 16), jnp.float32),
                  jax.ShapeDtypeStruct((_R,), jnp.int32)),
        scratch_types=[
            pltpu.VMEM((8, 128), jnp.int32),
            pltpu.VMEM((8, 128), jnp.int32),
            pltpu.VMEM((1024, 16), jnp.float32),
            pltpu.SemaphoreType.DMA,
        ],
        compiler_params=_SC_PARAMS,
    )
    def k(xyz_hbm, gall_hbm, g1_hbm, ar_hbm, xg_hbm, pos_hbm, gv, av, rows_v, sem):
        wid = lax.axis_index("s") * 2 + lax.axis_index("c")
        pltpu.sync_copy(g1_hbm.at[wid], gv)
        pltpu.sync_copy(ar_hbm.at[wid], av)
        hs = [pltpu.async_copy(av.at[j], pos_hbm.at[gv.at[j]], sem) for j in range(8)]
        for h in hs:
            h.wait()
        for job in range(3):
            jid = wid * 3 + job
            base = pl.multiple_of(jid * 1024, 1024)
            pltpu.sync_copy(gall_hbm.at[jid], gv)
            hs = [pltpu.async_copy(xyz_hbm.at[gv.at[j]],
                                   rows_v.at[pl.ds(j * 128, 128)], sem)
                  for j in range(8)]
            for h in hs:
                h.wait()
            pltpu.sync_copy(rows_v, xg_hbm.at[pl.ds(base, 1024)])

    return k(xyzrows, gall, g1c, arc)


def _sc_scatter_home(m23, g2c, g3c, pos):
    mesh = plsc.VectorSubcoreMesh(**_MESH)

    @functools.partial(
        pl.kernel, mesh=mesh,
        out_type=(jax.ShapeDtypeStruct((_R, 64), jnp.float32),
                  jax.ShapeDtypeStruct((_R, 64), jnp.float32)),
        scratch_types=[
            pltpu.VMEM((8, 128), jnp.int32),
            pltpu.VMEM((8, 128), jnp.int32),
            pltpu.VMEM((1024, 64), jnp.float32),
            pltpu.SemaphoreType.DMA,
        ],
        compiler_params=_SC_PARAMS,
    )
    def k(m_hbm, g2_hbm, g3_hbm, pos_hbm, mh2_hbm, mh3_hbm, gv, sv, rows_v, sem):
        wid = lax.axis_index("s") * 2 + lax.axis_index("c")
        for ci in range(2):
            g_hbm = (g2_hbm, g3_hbm)[ci]
            out_hbm = (mh2_hbm, mh3_hbm)[ci]
            base = pl.multiple_of(ci * _R + wid * 1024, 1024)
            pltpu.sync_copy(g_hbm.at[wid], gv)
            hs = [pltpu.async_copy(pos_hbm.at[gv.at[j]], sv.at[j], sem) for j in range(8)]
            for h in hs:
                h.wait()
            pltpu.sync_copy(m_hbm.at[pl.ds(base, 1024)], rows_v)
            hs = [pltpu.async_copy(rows_v.at[pl.ds(j * 128, 128)],
                                   out_hbm.at[sv.at[j]], sem)
                  for j in range(8)]
            for h in hs:
                h.wait()

    return k(m23, g2c, g3c, pos)


def _sc_gather_home(xin2, g2c, pos):
    mesh = plsc.VectorSubcoreMesh(**_MESH)

    @functools.partial(
        pl.kernel, mesh=mesh,
        out_type=jax.ShapeDtypeStruct((_R, 64), jnp.float32),
        scratch_types=[
            pltpu.VMEM((8, 128), jnp.int32),
            pltpu.VMEM((8, 128), jnp.int32),
            pltpu.VMEM((1024, 64), jnp.float32),
            pltpu.SemaphoreType.DMA,
        ],
        compiler_params=_SC_PARAMS,
    )
    def k(x_hbm, g2_hbm, pos_hbm, out_hbm, gv, sv, rows_v, sem):
        wid = lax.axis_index("s") * 2 + lax.axis_index("c")
        base = pl.multiple_of(wid * 1024, 1024)
        pltpu.sync_copy(g2_hbm.at[wid], gv)
        hs = [pltpu.async_copy(pos_hbm.at[gv.at[j]], sv.at[j], sem) for j in range(8)]
        for h in hs:
            h.wait()
        hs = [pltpu.async_copy(x_hbm.at[sv.at[j]],
                               rows_v.at[pl.ds(j * 128, 128)], sem)
              for j in range(8)]
        for h in hs:
            h.wait()
        pltpu.sync_copy(rows_v, out_hbm.at[pl.ds(base, 1024)])

    return k(xin2, g2c, pos)


def _sc_scatter_home128(m22, g2c4, pos):
    mesh = plsc.VectorSubcoreMesh(**_MESH)

    @functools.partial(
        pl.kernel, mesh=mesh,
        out_type=jax.ShapeDtypeStruct((_R, 128), jnp.float32),
        scratch_types=[
            pltpu.VMEM((4, 128), jnp.int32),
            pltpu.VMEM((4, 128), jnp.int32),
            pltpu.VMEM((512, 128), jnp.float32),
            pltpu.SemaphoreType.DMA,
        ],
        compiler_params=_SC_PARAMS,
    )
    def k(m_hbm, g2_hbm, pos_hbm, out_hbm, gv, sv, rows_v, sem):
        wid = lax.axis_index("s") * 2 + lax.axis_index("c")
        for job in range(2):
            jid = wid * 2 + job
            base = pl.multiple_of(jid * 512, 512)
            pltpu.sync_copy(g2_hbm.at[jid], gv)
            hs = [pltpu.async_copy(pos_hbm.at[gv.at[j]], sv.at[j], sem) for j in range(4)]
            for h in hs:
                h.wait()
            pltpu.sync_copy(m_hbm.at[pl.ds(base, 512)], rows_v)
            hs = [pltpu.async_copy(rows_v.at[pl.ds(j * 128, 128)],
                                   out_hbm.at[sv.at[j]], sem)
                  for j in range(4)]
            for h in hs:
                h.wait()

    return k(m22, g2c4, pos)


def _shift(x, s):
    return pltpu.roll(x, _N - s, 1)


def _slide_max(x, k):
    cur, w = x, 1
    while 2 * w <= k:
        cur = jnp.maximum(cur, _shift(cur, w))
        w *= 2
    if w < k:
        cur = jnp.maximum(cur, _shift(cur, k - w))
    return cur


def _branch_body(x_ref, wn_ref, o_ref, *, k):
    bv = jnp.dot(x_ref[...], wn_ref[...], preferred_element_type=jnp.float32)
    o = bv.shape[1]
    m = _slide_max(bv.reshape(_BN, _N, o), k)
    o_ref[...] = m.reshape(_BN * _N, o)


def _tc_branch(xg, wn, k, first_block, nblocks):
    w = xg.shape[1]
    o = wn.shape[1]
    blk = _BN * _N
    return pl.pallas_call(
        functools.partial(_branch_body, k=k),
        grid=(nblocks,),
        in_specs=[pl.BlockSpec((blk, w), lambda i: (i + first_block, 0)),
                  pl.BlockSpec((w, o), lambda i: (0, 0))],
        out_specs=pl.BlockSpec((blk, o), lambda i: (i, 0)),
        out_shape=jax.ShapeDtypeStruct((nblocks * blk, o), jnp.float32),
    )(xg, wn)


def _combine2_body(x_ref, mh2_ref, mh3_ref, wn1_ref, wd1_ref, b1_ref,
                   wcat2_ref, b2_ref, xin2_ref, a2_ref, m21_ref):
    xgb = x_ref[...]
    bv1 = jnp.dot(xgb, wn1_ref[...], preferred_element_type=jnp.float32)
    m1 = _slide_max(bv1.reshape(_BN, _N, 64), 24).reshape(_BN * _N, 64)
    a1 = jnp.dot(xgb, wd1_ref[...], preferred_element_type=jnp.float32) + b1_ref[...]
    m = jnp.maximum(m1, jnp.maximum(mh2_ref[...], mh3_ref[...]))
    xin2 = jnp.maximum(a1 + m, 0.0)
    e2 = jnp.dot(xin2, wcat2_ref[...], preferred_element_type=jnp.float32)
    a2 = e2[:, :128] + b2_ref[...]
    m21 = _slide_max(e2[:, 128:].reshape(_BN, _N, 128), 6).reshape(_BN * _N, 128)
    xin2_ref[...] = xin2
    a2_ref[...] = a2
    m21_ref[...] = m21


def _tc_combine2(xg, mh2, mh3, wn1, wd1, b1, wcat2, b2):
    blk = _BN * _N
    return pl.pallas_call(
        _combine2_body,
        grid=(_B // _BN,),
        in_specs=[pl.BlockSpec((blk, 16), lambda i: (i, 0)),
                  pl.BlockSpec((blk, 64), lambda i: (i, 0)),
                  pl.BlockSpec((blk, 64), lambda i: (i, 0)),
                  pl.BlockSpec((16, 64), lambda i: (0, 0)),
                  pl.BlockSpec((16, 64), lambda i: (0, 0)),
                  pl.BlockSpec((1, 64), lambda i: (0, 0)),
                  pl.BlockSpec((64, 256), lambda i: (0, 0)),
                  pl.BlockSpec((1, 128), lambda i: (0, 0))],
        out_specs=[pl.BlockSpec((blk, 64), lambda i: (i, 0)),
                   pl.BlockSpec((blk, 128), lambda i: (i, 0)),
                   pl.BlockSpec((blk, 128), lambda i: (i, 0))],
        out_shape=[jax.ShapeDtypeStruct((_R, 64), jnp.float32),
                   jax.ShapeDtypeStruct((_R, 128), jnp.float32),
                   jax.ShapeDtypeStruct((_R, 128), jnp.float32)],
    )(xg, mh2, mh3, wn1, wd1, b1, wcat2, b2)


def _head_body(a2_ref, m21_ref, m22_ref, wcat_ref, b3_ref, wfc_ref, bfc_ref, o_ref):
    xin3 = jnp.maximum(a2_ref[...] + jnp.maximum(m21_ref[...], m22_ref[...]), 0.0)
    e = jnp.dot(xin3, wcat_ref[...], preferred_element_type=jnp.float32)
    a3 = e[:, :256].reshape(_BN, _N, 256) + b3_ref[...]
    m3 = _slide_max(e[:, 256:].reshape(_BN, _N, 256), 6)
    y = jnp.maximum(a3 + m3, 0.0)
    g = jnp.max(y, axis=1)
    o_ref[...] = jnp.dot(g, wfc_ref[...], preferred_element_type=jnp.float32) + bfc_ref[...]


def _tc_head(a2h, m21h, m22h, wcat, b3, wfc, bfc):
    blk = _BN * _N
    return pl.pallas_call(
        _head_body,
        grid=(_B // _BN,),
        in_specs=[pl.BlockSpec((blk, 128), lambda i: (i, 0)),
                  pl.BlockSpec((blk, 128), lambda i: (i, 0)),
                  pl.BlockSpec((blk, 128), lambda i: (i, 0)),
                  pl.BlockSpec((128, 512), lambda i: (0, 0)),
                  pl.BlockSpec((1, 256), lambda i: (0, 0)),
                  pl.BlockSpec((256, 40), lambda i: (0, 0)),
                  pl.BlockSpec((1, 40), lambda i: (0, 0))],
        out_specs=pl.BlockSpec((_BN, 40), lambda i: (i, 0)),
        out_shape=jax.ShapeDtypeStruct((_B, 40), jnp.float32),
    )(a2h, m21h, m22h, wcat, b3, wfc, bfc)


def kernel(xyz, curve1, curve2, curve3, W1, b1, W2, b2, W3, b3, Wfc, bfc):
    f32 = jnp.float32
    offs = (jnp.arange(_B, dtype=jnp.int32) * _N)[:, None]
    g1 = (curve1 + offs).reshape(-1)
    g2 = (curve2 + offs).reshape(-1)
    g3 = (curve3 + offs).reshape(-1)

    gall = jnp.concatenate([g1, g2, g3]).reshape(-1, 8, 128)
    g1c = g1.reshape(-1, 8, 128)
    g2c = g2.reshape(-1, 8, 128)
    g3c = g3.reshape(-1, 8, 128)
    g2c4 = g2.reshape(-1, 4, 128)
    arc = jnp.arange(_R, dtype=jnp.int32).reshape(-1, 8, 128)

    xr = jnp.transpose(xyz, (0, 2, 1)).reshape(_R, 3)
    xr = jnp.pad(xr, ((0, 0), (0, 13))).astype(f32)

    wd1 = jnp.pad(W1[:3] - W1[3:], ((0, 13), (0, 0))).astype(f32)
    wn1 = jnp.pad(W1[3:], ((0, 13), (0, 0))).astype(f32)
    wd2, wn2 = (W2[:64] - W2[64:]).astype(f32), W2[64:].astype(f32)
    wcat2 = jnp.concatenate([wd2, wn2], axis=1)
    wd3, wn3 = (W3[:128] - W3[128:]).astype(f32), W3[128:].astype(f32)
    w3cat = jnp.concatenate([wd3, wn3], axis=1)

    xg, pos = _sc_stage1(xr, gall, g1c, arc)

    m23 = _tc_branch(xg, wn1, 24, first_block=4, nblocks=8)
    mh2, mh3 = _sc_scatter_home(m23, g2c, g3c, pos)

    xin2, a2h, m21h = _tc_combine2(xg, mh2, mh3, wn1, wd1,
                                   b1.reshape(1, -1).astype(f32),
                                   wcat2, b2.reshape(1, -1).astype(f32))

    xg22 = _sc_gather_home(xin2, g2c, pos)
    m22 = _tc_branch(xg22, wn2, 6, first_block=0, nblocks=4)
    m22h = _sc_scatter_home128(m22, g2c4, pos)

    return _tc_head(a2h, m21h, m22h, w3cat, b3.reshape(1, -1).astype(f32),
                    Wfc.astype(f32), bfc.reshape(1, -1).astype(f32))


# speedup vs baseline: 8.9515x; 1.0052x over previous
import functools

import jax
import jax.numpy as jnp
from jax import lax
from jax.experimental import pallas as pl
from jax.experimental.pallas import tpu as pltpu
from jax.experimental.pallas import tpu_sc as plsc

_B, _N = 32, 1024
_R = _B * _N
_NW = 32
_BN = 8

_SC_FLAT = pltpu.CompilerParams(use_tc_tiling_on_sc=False)
_SC_TILED = pltpu.CompilerParams(use_tc_tiling_on_sc=True)
_MESH = dict(core_axis_name="c", subcore_axis_name="s")


def _sc_stage1(xyzrows, gall, g1c, arc):
    mesh = plsc.VectorSubcoreMesh(**_MESH)

    @functools.partial(
        pl.kernel, mesh=mesh,
        out_type=(jax.ShapeDtypeStruct((3 * _R, 16), jnp.float32),
                  jax.ShapeDtypeStruct((_R,), jnp.int32)),
        scratch_types=[
            pltpu.VMEM((8, 128), jnp.int32),
            pltpu.VMEM((8, 128), jnp.int32),
            pltpu.VMEM((3, 8, 128), jnp.int32),
            pltpu.VMEM((3, 1024, 16), jnp.float32),
            pltpu.SemaphoreType.DMA,
        ],
        compiler_params=_SC_FLAT,
    )
    def k(xyz_hbm, gall_hbm, g1_hbm, ar_hbm, xg_hbm, pos_hbm, gv, av, gv3, rows_v, sem):
        wid = lax.axis_index("s") * 2 + lax.axis_index("c")
        pltpu.sync_copy(g1_hbm.at[wid], gv)
        pltpu.sync_copy(ar_hbm.at[wid], av)
        for job in range(3):
            pltpu.sync_copy(gall_hbm.at[wid * 3 + job], gv3.at[job])
        hs = [pltpu.async_copy(av.at[j], pos_hbm.at[gv.at[j]], sem) for j in range(8)]
        for job in range(3):
            hs += [pltpu.async_copy(xyz_hbm.at[gv3.at[job].at[j]],
                                    rows_v.at[job].at[pl.ds(j * 128, 128)], sem)
                   for j in range(8)]
        for h in hs:
            h.wait()
        for job in range(3):
            base = pl.multiple_of((wid * 3 + job) * 1024, 1024)
            pltpu.sync_copy(rows_v.at[job], xg_hbm.at[pl.ds(base, 1024)])

    return k(xyzrows, gall, g1c, arc)


def _sc_resolve(g2c, g3c, pos):
    mesh = plsc.VectorSubcoreMesh(**_MESH)

    @functools.partial(
        pl.kernel, mesh=mesh,
        out_type=(jax.ShapeDtypeStruct((_R // 1024, 8, 128), jnp.int32),
                  jax.ShapeDtypeStruct((_R // 1024, 8, 128), jnp.int32)),
        scratch_types=[
            pltpu.VMEM((8, 128), jnp.int32),
            pltpu.VMEM((2, 8, 128), jnp.int32),
            pltpu.SemaphoreType.DMA,
        ],
        compiler_params=_SC_FLAT,
    )
    def k(g2_hbm, g3_hbm, pos_hbm, s2_hbm, s3_hbm, gv, sv, sem):
        wid = lax.axis_index("s") * 2 + lax.axis_index("c")
        hs = []
        for ci in range(2):
            g_hbm = (g2_hbm, g3_hbm)[ci]
            pltpu.sync_copy(g_hbm.at[wid], gv)
            hs += [pltpu.async_copy(pos_hbm.at[gv.at[j]], sv.at[ci].at[j], sem)
                   for j in range(8)]
        for h in hs:
            h.wait()
        pltpu.sync_copy(sv.at[0], s2_hbm.at[wid])
        pltpu.sync_copy(sv.at[1], s3_hbm.at[wid])

    return k(g2c, g3c, pos)


def _sc_scatter2(src, s2c4, s3c4):
    mesh = plsc.VectorSubcoreMesh(**_MESH)

    @functools.partial(
        pl.kernel, mesh=mesh,
        out_type=(jax.ShapeDtypeStruct((_R, 128), jnp.float32),
                  jax.ShapeDtypeStruct((_R, 128), jnp.float32)),
        scratch_types=[
            pltpu.VMEM((4, 128), jnp.int32),
            pltpu.VMEM((512, 128), jnp.float32),
            pltpu.SemaphoreType.DMA,
        ],
        compiler_params=_SC_TILED,
    )
    def k(src_hbm, s2_hbm, s3_hbm, out2_hbm, out3_hbm, sv, rows_v, sem):
        wid = lax.axis_index("s") * 2 + lax.axis_index("c")
        for ci in range(2):
            s_hbm = (s2_hbm, s3_hbm)[ci]
            out_hbm = (out2_hbm, out3_hbm)[ci]
            for job in range(2):
                jid = wid * 2 + job
                base = pl.multiple_of(ci * _R + jid * 512, 512)
                pltpu.sync_copy(s_hbm.at[jid], sv)
                pltpu.sync_copy(src_hbm.at[pl.ds(base, 512)], rows_v)
                hs = [pltpu.async_copy(rows_v.at[pl.ds(j * 128, 128)],
                                       out_hbm.at[sv.at[j]], sem)
                      for j in range(4)]
                for h in hs:
                    h.wait()

    return k(src, s2c4, s3c4)


def _sc_gather1(table, s2c4):
    mesh = plsc.VectorSubcoreMesh(**_MESH)

    @functools.partial(
        pl.kernel, mesh=mesh,
        out_type=jax.ShapeDtypeStruct((_R, 128), jnp.float32),
        scratch_types=[
            pltpu.VMEM((4, 128), jnp.int32),
            pltpu.VMEM((512, 128), jnp.float32),
            pltpu.SemaphoreType.DMA,
        ],
        compiler_params=_SC_TILED,
    )
    def k(t_hbm, s2_hbm, out_hbm, sv, rows_v, sem):
        wid = lax.axis_index("s") * 2 + lax.axis_index("c")
        for job in range(2):
            jid = wid * 2 + job
            base = pl.multiple_of(jid * 512, 512)
            pltpu.sync_copy(s2_hbm.at[jid], sv)
            hs = [pltpu.async_copy(t_hbm.at[sv.at[j]],
                                   rows_v.at[pl.ds(j * 128, 128)], sem)
                  for j in range(4)]
            for h in hs:
                h.wait()
            pltpu.sync_copy(rows_v, out_hbm.at[pl.ds(base, 512)])

    return k(table, s2c4)


def _sc_scatter1(src, s2c4):
    mesh = plsc.VectorSubcoreMesh(**_MESH)

    @functools.partial(
        pl.kernel, mesh=mesh,
        out_type=jax.ShapeDtypeStruct((_R, 128), jnp.float32),
        scratch_types=[
            pltpu.VMEM((4, 128), jnp.int32),
            pltpu.VMEM((512, 128), jnp.float32),
            pltpu.SemaphoreType.DMA,
        ],
        compiler_params=_SC_TILED,
    )
    def k(src_hbm, s2_hbm, out_hbm, sv, rows_v, sem):
        wid = lax.axis_index("s") * 2 + lax.axis_index("c")
        for job in range(2):
            jid = wid * 2 + job
            base = pl.multiple_of(jid * 512, 512)
            pltpu.sync_copy(s2_hbm.at[jid], sv)
            pltpu.sync_copy(src_hbm.at[pl.ds(base, 512)], rows_v)
            hs = [pltpu.async_copy(rows_v.at[pl.ds(j * 128, 128)],
                                   out_hbm.at[sv.at[j]], sem)
                  for j in range(4)]
            for h in hs:
                h.wait()

    return k(src, s2c4)


def _shift(x, s):
    return pltpu.roll(x, _N - s, 1)


def _slide_max(x, k):
    cur, w = x, 1
    while 2 * w <= k:
        cur = jnp.maximum(cur, _shift(cur, w))
        w *= 2
    if w < k:
        cur = jnp.maximum(cur, _shift(cur, k - w))
    return cur


def _l1branch_body(x_ref, wcat_ref, b1_ref, o_ref):
    e = jnp.dot(x_ref[...], wcat_ref[...], preferred_element_type=jnp.float32)
    a = e[:, :64] + b1_ref[...]
    m = _slide_max(e[:, 64:].reshape(_BN, _N, 64), 24).reshape(_BN * _N, 64)
    o_ref[...] = jnp.concatenate([a, m], axis=1)


def _tc_l1branch(xg, wcat1, b1):
    blk = _BN * _N
    return pl.pallas_call(
        _l1branch_body,
        grid=(8,),
        in_specs=[pl.BlockSpec((blk, 16), lambda i: (i + 4, 0)),
                  pl.BlockSpec((16, 128), lambda i: (0, 0)),
                  pl.BlockSpec((1, 64), lambda i: (0, 0))],
        out_specs=pl.BlockSpec((blk, 128), lambda i: (i, 0)),
        out_shape=jax.ShapeDtypeStruct((2 * _R, 128), jnp.float32),
    )(xg, wcat1, b1)


def _slidemax_body(x_ref, o_ref, *, k):
    o_ref[...] = _slide_max(x_ref[...].reshape(_BN, _N, 128), k).reshape(_BN * _N, 128)


def _tc_slidemax(x, k):
    blk = _BN * _N
    return pl.pallas_call(
        functools.partial(_slidemax_body, k=k),
        grid=(_B // _BN,),
        in_specs=[pl.BlockSpec((blk, 128), lambda i: (i, 0))],
        out_specs=pl.BlockSpec((blk, 128), lambda i: (i, 0)),
        out_shape=jax.ShapeDtypeStruct((_R, 128), jnp.float32),
    )(x)


def _combine2_body(x_ref, sh2_ref, sh3_ref, wn1_ref, wcat2_ref, b2_ref,
                   a2_ref, m21_ref, bv2_ref):
    xgb = x_ref[...]
    bv1 = jnp.dot(xgb, wn1_ref[...], preferred_element_type=jnp.float32)
    m1 = _slide_max(bv1.reshape(_BN, _N, 64), 24).reshape(_BN * _N, 64)
    sh2 = sh2_ref[...]
    sh3 = sh3_ref[...]
    a1 = sh2[:, :64]
    m = jnp.maximum(m1, jnp.maximum(sh2[:, 64:], sh3[:, 64:]))
    xin2 = jnp.maximum(a1 + m, 0.0)
    e2 = jnp.dot(xin2, wcat2_ref[...], preferred_element_type=jnp.float32)
    a2_ref[...] = e2[:, :128] + b2_ref[...]
    m21_ref[...] = _slide_max(e2[:, 128:].reshape(_BN, _N, 128), 6).reshape(_BN * _N, 128)
    bv2_ref[...] = e2[:, 128:]


def _tc_combine2(xg, sh2, sh3, wn1, wcat2, b2):
    blk = _BN * _N
    return pl.pallas_call(
        _combine2_body,
        grid=(_B // _BN,),
        in_specs=[pl.BlockSpec((blk, 16), lambda i: (i, 0)),
                  pl.BlockSpec((blk, 128), lambda i: (i, 0)),
                  pl.BlockSpec((blk, 128), lambda i: (i, 0)),
                  pl.BlockSpec((16, 64), lambda i: (0, 0)),
                  pl.BlockSpec((64, 256), lambda i: (0, 0)),
                  pl.BlockSpec((1, 128), lambda i: (0, 0))],
        out_specs=[pl.BlockSpec((blk, 128), lambda i: (i, 0)),
                   pl.BlockSpec((blk, 128), lambda i: (i, 0)),
                   pl.BlockSpec((blk, 128), lambda i: (i, 0))],
        out_shape=[jax.ShapeDtypeStruct((_R, 128), jnp.float32),
                   jax.ShapeDtypeStruct((_R, 128), jnp.float32),
                   jax.ShapeDtypeStruct((_R, 128), jnp.float32)],
    )(xg, sh2, sh3, wn1, wcat2, b2)


def _head_body(a2_ref, m21_ref, m22_ref, wcat_ref, b3_ref, wfc_ref, bfc_ref, o_ref):
    xin3 = jnp.maximum(a2_ref[...] + jnp.maximum(m21_ref[...], m22_ref[...]), 0.0)
    e = jnp.dot(xin3, wcat_ref[...], preferred_element_type=jnp.float32)
    a3 = e[:, :256].reshape(_BN, _N, 256) + b3_ref[...]
    m3 = _slide_max(e[:, 256:].reshape(_BN, _N, 256), 6)
    y = jnp.maximum(a3 + m3, 0.0)
    g = jnp.max(y, axis=1)
    o_ref[...] = jnp.dot(g, wfc_ref[...], preferred_element_type=jnp.float32) + bfc_ref[...]


def _tc_head(a2h, m21h, m22h, wcat, b3, wfc, bfc):
    blk = _BN * _N
    return pl.pallas_call(
        _head_body,
        grid=(_B // _BN,),
        in_specs=[pl.BlockSpec((blk, 128), lambda i: (i, 0)),
                  pl.BlockSpec((blk, 128), lambda i: (i, 0)),
                  pl.BlockSpec((blk, 128), lambda i: (i, 0)),
                  pl.BlockSpec((128, 512), lambda i: (0, 0)),
                  pl.BlockSpec((1, 256), lambda i: (0, 0)),
                  pl.BlockSpec((256, 40), lambda i: (0, 0)),
                  pl.BlockSpec((1, 40), lambda i: (0, 0))],
        out_specs=pl.BlockSpec((_BN, 40), lambda i: (i, 0)),
        out_shape=jax.ShapeDtypeStruct((_B, 40), jnp.float32),
    )(a2h, m21h, m22h, wcat, b3, wfc, bfc)


def kernel(xyz, curve1, curve2, curve3, W1, b1, W2, b2, W3, b3, Wfc, bfc):
    f32 = jnp.float32
    offs = (jnp.arange(_B, dtype=jnp.int32) * _N)[:, None]
    g1 = (curve1 + offs).reshape(-1)
    g2 = (curve2 + offs).reshape(-1)
    g3 = (curve3 + offs).reshape(-1)

    gall = jnp.concatenate([g1, g2, g3]).reshape(-1, 8, 128)
    g1c = g1.reshape(-1, 8, 128)
    g2c = g2.reshape(-1, 8, 128)
    g3c = g3.reshape(-1, 8, 128)
    arc = jnp.arange(_R, dtype=jnp.int32).reshape(-1, 8, 128)

    xr = jnp.transpose(xyz, (0, 2, 1)).reshape(_R, 3)
    xr = jnp.pad(xr, ((0, 0), (0, 13))).astype(f32)

    wd1 = jnp.pad(W1[:3] - W1[3:], ((0, 13), (0, 0))).astype(f32)
    wn1 = jnp.pad(W1[3:], ((0, 13), (0, 0))).astype(f32)
    wcat1 = jnp.concatenate([wd1, wn1], axis=1)
    wd2, wn2 = (W2[:64] - W2[64:]).astype(f32), W2[64:].astype(f32)
    wcat2 = jnp.concatenate([wd2, wn2], axis=1)
    wd3, wn3 = (W3[:128] - W3[128:]).astype(f32), W3[128:].astype(f32)
    w3cat = jnp.concatenate([wd3, wn3], axis=1)

    xg, pos = _sc_stage1(xr, gall, g1c, arc)
    s2c, s3c = _sc_resolve(g2c, g3c, pos)
    s2c4 = s2c.reshape(-1, 4, 128)
    s3c4 = s3c.reshape(-1, 4, 128)

    s23 = _tc_l1branch(xg, wcat1, b1.reshape(1, -1).astype(f32))
    sh2, sh3 = _sc_scatter2(s23, s2c4, s3c4)

    a2h, m21h, bv2h = _tc_combine2(xg, sh2, sh3, wn1, wcat2,
                                   b2.reshape(1, -1).astype(f32))

    bg22 = _sc_gather1(bv2h, s2c4)
    m22 = _tc_slidemax(bg22, 6)
    m22h = _sc_scatter1(m22, s2c4)

    return _tc_head(a2h, m21h, m22h, w3cat, b3.reshape(1, -1).astype(f32),
                    Wfc.astype(f32), bfc.reshape(1, -1).astype(f32))
